# baseline (speedup 1.0000x reference)
import jax
import jax.numpy as jnp
from jax.experimental import pallas as pl

_ROWS_PER_BLOCK = 1024
_CHUNK = 256


def _vq_block(tmp_ref, emb_ref, q_ref, ste_ref, idx_ref):
    tmp = tmp_ref[...]
    emb = emb_ref[...]
    rows = tmp.shape[0]
    k = emb.shape[0]
    s2 = jnp.sum(tmp * tmp, axis=1, keepdims=True)
    e2 = jnp.sum(emb * emb, axis=1)[None, :]
    ab = jax.lax.dot_general(
        tmp, emb, (((1,), (1,)), ((), ())),
        preferred_element_type=jnp.float32,
        precision=jax.lax.Precision.DEFAULT)
    best_d = jnp.full((rows, _CHUNK), jnp.inf, dtype=jnp.float32)
    best_i = jnp.zeros((rows, _CHUNK), dtype=jnp.int32)
    for ci in range(k // _CHUNK):
        c0 = ci * _CHUNK
        d = (s2 - 2.0 * ab[:, c0:c0 + _CHUNK]) + e2[:, c0:c0 + _CHUNK]
        iota = jax.lax.broadcasted_iota(jnp.int32, (rows, _CHUNK), 1) + c0
        m = d < best_d
        best_i = jnp.where(m, iota, best_i)
        best_d = jnp.where(m, d, best_d)
    mind = jnp.min(best_d, axis=1, keepdims=True)
    cand = jnp.where(best_d == mind, best_i, k)
    idx = jnp.min(cand, axis=1)
    iota_k = jax.lax.broadcasted_iota(jnp.int32, (rows, k), 1)
    oh = (iota_k == idx[:, None]).astype(jnp.float32)
    q = jax.lax.dot_general(
        oh, emb, (((1,), (0,)), ((), ())),
        preferred_element_type=jnp.float32,
        precision=jax.lax.Precision.HIGHEST)
    q_ref[...] = q
    ste_ref[...] = q
    idx_ref[...] = idx[:, None]


def kernel(z, emb):
    n, c, h, w = z.shape
    k = emb.shape[0]
    s_total = n * h * w
    tmp = jnp.transpose(z, (0, 2, 3, 1)).reshape(s_total, c)
    blk = _ROWS_PER_BLOCK
    q, ste, idx = pl.pallas_call(
        _vq_block,
        grid=(s_total // blk,),
        in_specs=[
            pl.BlockSpec((blk, c), lambda b: (b, 0)),
            pl.BlockSpec((k, c), lambda b: (0, 0)),
        ],
        out_specs=[
            pl.BlockSpec((blk, c), lambda b: (b, 0)),
            pl.BlockSpec((blk, c), lambda b: (b, 0)),
            pl.BlockSpec((blk, 1), lambda b: (b, 0)),
        ],
        out_shape=[
            jax.ShapeDtypeStruct((s_total, c), jnp.float32),
            jax.ShapeDtypeStruct((s_total, c), jnp.float32),
            jax.ShapeDtypeStruct((s_total, 1), jnp.int32),
        ],
    )(tmp, emb)
    quantized = q.reshape(n, h, w, c).transpose(0, 3, 1, 2)
    ste_out = ste.reshape(n, h, w, c).transpose(0, 3, 1, 2)
    indxs = idx.reshape(n, h, w)
    return (quantized, ste_out, indxs)


# speedup vs baseline: 1.3748x; 1.3748x over previous
import jax
import jax.numpy as jnp
from jax.experimental import pallas as pl
from jax.experimental.pallas import tpu as pltpu

_ROWS_PER_BLOCK = 1024
_CHUNK = 128


def _vq_block(tmp_ref, emb_ref, q_ref, ste_ref, idx_ref, e2_ref):
    tmp = tmp_ref[...]
    emb = emb_ref[...]
    rows = tmp.shape[0]
    k = emb.shape[0]

    @pl.when(pl.program_id(0) == 0)
    def _():
        e2_ref[...] = jnp.sum(emb * emb, axis=1)[None, :]

    s2 = jnp.sum(tmp * tmp, axis=1, keepdims=True)
    e2 = e2_ref[...]
    ab2 = jax.lax.dot_general(
        2.0 * tmp, emb, (((1,), (1,)), ((), ())),
        preferred_element_type=jnp.float32,
        precision=jax.lax.Precision.DEFAULT)
    best_d = jnp.full((rows, _CHUNK), jnp.inf, dtype=jnp.float32)
    best_i = jnp.zeros((rows, _CHUNK), dtype=jnp.int32)
    for ci in range(k // _CHUNK):
        c0 = ci * _CHUNK
        d = (s2 - ab2[:, c0:c0 + _CHUNK]) + e2[:, c0:c0 + _CHUNK]
        iota = jax.lax.broadcasted_iota(jnp.int32, (rows, _CHUNK), 1) + c0
        m = d < best_d
        best_i = jnp.where(m, iota, best_i)
        best_d = jnp.where(m, d, best_d)
    mind = jnp.min(best_d, axis=1, keepdims=True)
    cand = jnp.where(best_d == mind, best_i, k)
    idx = jnp.min(cand, axis=1)

    iota_k = jax.lax.broadcasted_iota(jnp.int32, (rows, k), 1)
    oh = (iota_k == idx[:, None]).astype(jnp.bfloat16)
    dims = (((1,), (0,)), ((), ()))
    e_hi = emb.astype(jnp.bfloat16)
    r1 = emb - e_hi.astype(jnp.float32)
    e_mid = r1.astype(jnp.bfloat16)
    e_lo = (r1 - e_mid.astype(jnp.float32)).astype(jnp.bfloat16)
    q = ((jax.lax.dot_general(oh, e_hi, dims,
                              preferred_element_type=jnp.float32)
          + jax.lax.dot_general(oh, e_mid, dims,
                                preferred_element_type=jnp.float32))
         + jax.lax.dot_general(oh, e_lo, dims,
                               preferred_element_type=jnp.float32))
    q_ref[...] = q
    ste_ref[...] = q
    idx_ref[...] = idx[:, None]


def kernel(z, emb):
    n, c, h, w = z.shape
    k = emb.shape[0]
    s_total = n * h * w
    tmp = jnp.transpose(z, (0, 2, 3, 1)).reshape(s_total, c)
    blk = _ROWS_PER_BLOCK
    q, ste, idx = pl.pallas_call(
        _vq_block,
        grid=(s_total // blk,),
        in_specs=[
            pl.BlockSpec((blk, c), lambda b: (b, 0)),
            pl.BlockSpec((k, c), lambda b: (0, 0)),
        ],
        out_specs=[
            pl.BlockSpec((blk, c), lambda b: (b, 0)),
            pl.BlockSpec((blk, c), lambda b: (b, 0)),
            pl.BlockSpec((blk, 1), lambda b: (b, 0)),
        ],
        out_shape=[
            jax.ShapeDtypeStruct((s_total, c), jnp.float32),
            jax.ShapeDtypeStruct((s_total, c), jnp.float32),
            jax.ShapeDtypeStruct((s_total, 1), jnp.int32),
        ],
        scratch_shapes=[pltpu.VMEM((1, k), jnp.float32)],
    )(tmp, emb)
    quantized = q.reshape(n, h, w, c).transpose(0, 3, 1, 2)
    ste_out = ste.reshape(n, h, w, c).transpose(0, 3, 1, 2)
    indxs = idx.reshape(n, h, w)
    return (quantized, ste_out, indxs)


# speedup vs baseline: 1.5766x; 1.1468x over previous
import jax
import jax.numpy as jnp
from jax.experimental import pallas as pl

_K = 1024
_ROWS_PER_BLOCK = 1024


def _vq_block(tmp_ref, emb_ref, q_ref, ste_ref, idx_ref):
    tmp = tmp_ref[...]
    emb = emb_ref[...]
    s2 = jnp.sum(tmp * tmp, axis=1, keepdims=True)
    e2 = jnp.sum(emb * emb, axis=1)[None, :]
    ab = jax.lax.dot_general(
        tmp, emb, (((1,), (1,)), ((), ())),
        preferred_element_type=jnp.float32,
        precision=jax.lax.Precision.DEFAULT)
    d = (s2 - 2.0 * ab) + e2
    mind = jnp.min(d, axis=1, keepdims=True)
    iota = jax.lax.broadcasted_iota(jnp.int32, d.shape, 1)
    idx = jnp.min(jnp.where(d == mind, iota, _K), axis=1)
    oh = (iota == idx[:, None]).astype(jnp.bfloat16)
    dims = (((0,), (1,)), ((), ()))
    e_hi = emb.astype(jnp.bfloat16)
    r1 = emb - e_hi.astype(jnp.float32)
    e_mid = r1.astype(jnp.bfloat16)
    q_cs = (jax.lax.dot_general(e_hi, oh, dims,
                                preferred_element_type=jnp.float32)
            + jax.lax.dot_general(e_mid, oh, dims,
                                  preferred_element_type=jnp.float32))
    q = q_cs.T
    q_ref[...] = q
    ste_ref[...] = q
    idx_ref[...] = idx[:, None]


def kernel(z, emb):
    n, c, h, w = z.shape
    k = emb.shape[0]
    s_total = n * h * w
    tmp = jnp.transpose(z, (0, 2, 3, 1)).reshape(s_total, c)
    blk = _ROWS_PER_BLOCK
    nblk = s_total // blk
    q, ste, idx = pl.pallas_call(
        _vq_block,
        grid=(nblk,),
        in_specs=[
            pl.BlockSpec((blk, c), lambda b: (b, 0)),
            pl.BlockSpec((k, c), lambda b: (0, 0)),
        ],
        out_specs=[
            pl.BlockSpec((blk, c), lambda b: (b, 0)),
            pl.BlockSpec((blk, c), lambda b: (b, 0)),
            pl.BlockSpec((blk, 1), lambda b: (b, 0)),
        ],
        out_shape=[
            jax.ShapeDtypeStruct((s_total, c), jnp.float32),
            jax.ShapeDtypeStruct((s_total, c), jnp.float32),
            jax.ShapeDtypeStruct((s_total, 1), jnp.int32),
        ],
    )(tmp, emb)
    quantized = q.reshape(n, h, w, c).transpose(0, 3, 1, 2)
    ste_out = ste.reshape(n, h, w, c).transpose(0, 3, 1, 2)
    indxs = idx.reshape(n, h, w)
    return (quantized, ste_out, indxs)


# speedup vs baseline: 1.8670x; 1.1842x over previous
import jax
import jax.numpy as jnp
from jax.experimental import pallas as pl

_K = 1024
_ROWS_PER_BLOCK = 1024


def _vq_block(tmp_ref, emb_ref, q_ref, ste_ref, idx_ref):
    tmp = tmp_ref[...]
    emb = emb_ref[...]
    s2 = jnp.sum(tmp * tmp, axis=1, keepdims=True)
    e2 = jnp.sum(emb * emb, axis=1)[None, :]
    ab = jax.lax.dot_general(
        tmp, emb, (((1,), (1,)), ((), ())),
        preferred_element_type=jnp.float32,
        precision=jax.lax.Precision.DEFAULT)
    d = (s2 - 2.0 * ab) + e2
    mind = jnp.min(d, axis=1, keepdims=True)
    iota = jax.lax.broadcasted_iota(jnp.int32, d.shape, 1)
    idx = jnp.min(jnp.where(d == mind, iota, _K), axis=1)
    oh = (iota == idx[:, None]).astype(jnp.float32)
    q_cs = jax.lax.dot_general(
        emb, oh, (((0,), (1,)), ((), ())),
        preferred_element_type=jnp.float32)
    q = q_cs.T
    q_ref[...] = q
    ste_ref[...] = q
    idx_ref[...] = idx[:, None]


def kernel(z, emb):
    n, c, h, w = z.shape
    k = emb.shape[0]
    s_total = n * h * w
    tmp = jnp.transpose(z, (0, 2, 3, 1)).reshape(s_total, c)
    blk = _ROWS_PER_BLOCK
    nblk = s_total // blk
    q, ste, idx = pl.pallas_call(
        _vq_block,
        grid=(nblk,),
        in_specs=[
            pl.BlockSpec((blk, c), lambda b: (b, 0)),
            pl.BlockSpec((k, c), lambda b: (0, 0)),
        ],
        out_specs=[
            pl.BlockSpec((blk, c), lambda b: (b, 0)),
            pl.BlockSpec((blk, c), lambda b: (b, 0)),
            pl.BlockSpec((blk, 1), lambda b: (b, 0)),
        ],
        out_shape=[
            jax.ShapeDtypeStruct((s_total, c), jnp.float32),
            jax.ShapeDtypeStruct((s_total, c), jnp.float32),
            jax.ShapeDtypeStruct((s_total, 1), jnp.int32),
        ],
    )(tmp, emb)
    quantized = q.reshape(n, h, w, c).transpose(0, 3, 1, 2)
    ste_out = ste.reshape(n, h, w, c).transpose(0, 3, 1, 2)
    indxs = idx.reshape(n, h, w)
    return (quantized, ste_out, indxs)


# speedup vs baseline: 1.9546x; 1.0469x over previous
import jax
import jax.numpy as jnp
from jax.experimental import pallas as pl

_K = 1024
_ROWS_PER_BLOCK = 2048


def _vq_block(tmp_ref, emb_ref, q_ref, ste_ref, idx_ref):
    tmp = tmp_ref[...]
    emb = emb_ref[...]
    s2 = jnp.sum(tmp * tmp, axis=1, keepdims=True)
    e2 = jnp.sum(emb * emb, axis=1)[None, :]
    ab = jax.lax.dot_general(
        tmp, emb, (((1,), (1,)), ((), ())),
        preferred_element_type=jnp.float32,
        precision=jax.lax.Precision.DEFAULT)
    d = (s2 - 2.0 * ab) + e2
    mind = jnp.min(d, axis=1, keepdims=True)
    iota = jax.lax.broadcasted_iota(jnp.int32, d.shape, 1)
    idx = jnp.min(jnp.where(d == mind, iota, _K), axis=1)
    oh = (iota == idx[:, None]).astype(jnp.float32)
    q_cs = jax.lax.dot_general(
        emb, oh, (((0,), (1,)), ((), ())),
        preferred_element_type=jnp.float32)
    q = q_cs.T
    q_ref[...] = q
    ste_ref[...] = q
    idx_ref[...] = idx[:, None]


def kernel(z, emb):
    n, c, h, w = z.shape
    k = emb.shape[0]
    s_total = n * h * w
    tmp = jnp.transpose(z, (0, 2, 3, 1)).reshape(s_total, c)
    blk = _ROWS_PER_BLOCK
    nblk = s_total // blk
    q, ste, idx = pl.pallas_call(
        _vq_block,
        grid=(nblk,),
        in_specs=[
            pl.BlockSpec((blk, c), lambda b: (b, 0)),
            pl.BlockSpec((k, c), lambda b: (0, 0)),
        ],
        out_specs=[
            pl.BlockSpec((blk, c), lambda b: (b, 0)),
            pl.BlockSpec((blk, c), lambda b: (b, 0)),
            pl.BlockSpec((blk, 1), lambda b: (b, 0)),
        ],
        out_shape=[
            jax.ShapeDtypeStruct((s_total, c), jnp.float32),
            jax.ShapeDtypeStruct((s_total, c), jnp.float32),
            jax.ShapeDtypeStruct((s_total, 1), jnp.int32),
        ],
    )(tmp, emb)
    quantized = q.reshape(n, h, w, c).transpose(0, 3, 1, 2)
    ste_out = ste.reshape(n, h, w, c).transpose(0, 3, 1, 2)
    indxs = idx.reshape(n, h, w)
    return (quantized, ste_out, indxs)


# speedup vs baseline: 1.9567x; 1.0011x over previous
import jax
import jax.numpy as jnp
from jax.experimental import pallas as pl

_ROWS_PER_BLOCK = 2048


def _vq_block(tmp_ref, emb_ref, q_ref, ste_ref, idx_ref):
    tmp = tmp_ref[...]
    emb = emb_ref[...]
    k = emb.shape[0]
    s2 = jnp.sum(tmp * tmp, axis=1, keepdims=True)
    e2 = jnp.sum(emb * emb, axis=1)[None, :]
    ab = jax.lax.dot_general(
        tmp, emb, (((1,), (1,)), ((), ())),
        preferred_element_type=jnp.float32,
        precision=jax.lax.Precision.DEFAULT)
    d = (s2 - 2.0 * ab) + e2
    mind = jnp.min(d, axis=1, keepdims=True)
    iota = jax.lax.broadcasted_iota(jnp.int32, d.shape, 1)
    idx = jnp.min(jnp.where(d == mind, iota, k), axis=1)
    oh = (iota == idx[:, None]).astype(jnp.float32)
    q_cs = jax.lax.dot_general(
        emb, oh, (((0,), (1,)), ((), ())),
        preferred_element_type=jnp.float32)
    q = q_cs.T
    q_ref[...] = q
    ste_ref[...] = q
    idx_ref[...] = idx[:, None]


def kernel(z, emb):
    n, c, h, w = z.shape
    k = emb.shape[0]
    s_total = n * h * w
    tmp = jnp.transpose(z, (0, 2, 3, 1)).reshape(s_total, c)
    blk = _ROWS_PER_BLOCK
    q, ste, idx = pl.pallas_call(
        _vq_block,
        grid=(s_total // blk,),
        in_specs=[
            pl.BlockSpec((blk, c), lambda b: (b, 0)),
            pl.BlockSpec((k, c), lambda b: (0, 0)),
        ],
        out_specs=[
            pl.BlockSpec((blk, c), lambda b: (b, 0)),
            pl.BlockSpec((blk, c), lambda b: (b, 0)),
            pl.BlockSpec((blk, 1), lambda b: (b, 0)),
        ],
        out_shape=[
            jax.ShapeDtypeStruct((s_total, c), jnp.float32),
            jax.ShapeDtypeStruct((s_total, c), jnp.float32),
            jax.ShapeDtypeStruct((s_total, 1), jnp.int32),
        ],
    )(tmp, emb)
    quantized = q.reshape(n, h, w, c).transpose(0, 3, 1, 2)
    ste_out = ste.reshape(n, h, w, c).transpose(0, 3, 1, 2)
    indxs = idx.reshape(n, h, w)
    return (quantized, ste_out, indxs)
